# baseline (speedup 1.0000x reference)
import functools

import jax
import jax.numpy as jnp
from jax import lax
from jax.experimental import pallas as pl
from jax.experimental.pallas import tpu as pltpu
from jax.experimental.pallas import tpu_sc as plsc

NT = 50
NB = 16384
NUM_ROWS = NT * NB
DIM = 32
NC, NS = 2, 16
NW = NC * NS
B_PER_W = NUM_ROWS // NW
CH = 128
N_CHUNKS = B_PER_W // CH
K = 10
GROUP = K * CH
N_GROUPS = N_CHUNKS // K

NV = 1000000
CK = 160
NCK = NV // CK
NIT = (NCK + NW - 1) // NW

_mesh = plsc.VectorSubcoreMesh(
    core_axis_name="c", subcore_axis_name="s", num_cores=NC, num_subcores=NS
)


@functools.partial(
    pl.kernel,
    out_type=jax.ShapeDtypeStruct((NV // 4, 4 * DIM), jnp.float32),
    mesh=_mesh,
    compiler_params=pltpu.CompilerParams(use_tc_tiling_on_sc=True),
    scratch_types=[
        pltpu.VMEM((CK, DIM), jnp.float32),
        pltpu.VMEM((CK // 4, 4 * DIM), jnp.float32),
    ],
)
def _sc_detile(w_hbm, out_hbm, vm2, vml):
    wid = lax.axis_index("s") * NC + lax.axis_index("c")

    @pl.loop(0, NIT)
    def _i(i):
        c = i * NW + wid

        @pl.when(c < NCK)
        def _():
            pltpu.sync_copy(w_hbm.at[pl.ds(c * CK, CK)], vm2)

            @pl.loop(0, CK)
            def _j(j):
                x0 = vm2[j, pl.ds(0, 16)]
                x1 = vm2[j, pl.ds(16, 16)]
                r = j // 4
                col = (j % 4) * DIM
                vml[r, pl.ds(col, 16)] = x0
                vml[r, pl.ds(col + 16, 16)] = x1

            pltpu.sync_copy(vml, out_hbm.at[pl.ds(c * (CK // 4), CK // 4)])


@functools.partial(
    pl.kernel,
    out_type=jax.ShapeDtypeStruct((NUM_ROWS, DIM), jnp.float32),
    mesh=_mesh,
    compiler_params=pltpu.CompilerParams(use_tc_tiling_on_sc=False),
    scratch_types=[
        pltpu.VMEM((N_CHUNKS, CH), jnp.int32),
        pltpu.VMEM((GROUP, DIM), jnp.float32),
        pltpu.VMEM((GROUP, DIM), jnp.float32),
        pltpu.SemaphoreType.DMA,
        pltpu.SemaphoreType.DMA,
        pltpu.SemaphoreType.DMA,
        pltpu.SemaphoreType.DMA,
    ],
)
def _sc_gather(w_hbm, idx_hbm, out_hbm, idx_v, rows0, rows1, gs0, gs1, os0, os1):
    wid = lax.axis_index("s") * NC + lax.axis_index("c")
    base = wid * B_PER_W
    pltpu.sync_copy(idx_hbm.at[wid], idx_v)

    def fire(g, rows_v, sem):
        for j in range(K):
            pltpu.async_copy(
                w_hbm.at[idx_v.at[g * K + j]], rows_v.at[pl.ds(j * CH, CH)], sem
            )

    def wait_gathers(rows_v, sem):
        for j in range(K):
            pltpu.make_async_copy(
                w_hbm.at[idx_v.at[j]], rows_v.at[pl.ds(j * CH, CH)], sem
            ).wait()

    def start_out(g, rows_v, sem):
        pltpu.async_copy(rows_v, out_hbm.at[pl.ds(base + g * GROUP, GROUP)], sem)

    def wait_out(rows_v, sem):
        pltpu.make_async_copy(rows_v, out_hbm.at[pl.ds(base, GROUP)], sem).wait()

    fire(0, rows0, gs0)

    @pl.loop(0, N_GROUPS, step=2)
    def _pair(g):
        @pl.when(g > 0)
        def _():
            wait_out(rows1, os1)
        fire(g + 1, rows1, gs1)
        wait_gathers(rows0, gs0)
        start_out(g, rows0, os0)

        @pl.when(g + 2 < N_GROUPS)
        def _():
            wait_out(rows0, os0)
            fire(g + 2, rows0, gs0)
        wait_gathers(rows1, gs1)
        start_out(g + 1, rows1, os1)

    wait_out(rows0, os0)
    wait_out(rows1, os1)


def kernel(ids, weight):
    w_lin = _sc_detile(weight).reshape(NV, DIM)
    idx = ids.T.astype(jnp.int32).reshape(NW, N_CHUNKS, CH)
    out = _sc_gather(w_lin, idx)
    return out.reshape(NT, NB, DIM).transpose(1, 0, 2)


# speedup vs baseline: 1.3295x; 1.3295x over previous
import functools

import jax
import jax.numpy as jnp
from jax import lax
from jax.experimental import pallas as pl
from jax.experimental.pallas import tpu as pltpu
from jax.experimental.pallas import tpu_sc as plsc

NT = 50
NB = 16384
NUM_ROWS = NT * NB
DIM = 32
NC, NS = 2, 16
NW = NC * NS
B_PER_W = NUM_ROWS // NW
CH = 128
N_CHUNKS = B_PER_W // CH
K = 10
GROUP = K * CH
N_GROUPS = N_CHUNKS // K

assert NW * B_PER_W == NUM_ROWS and CH * N_CHUNKS == B_PER_W
assert K * N_GROUPS == N_CHUNKS and N_GROUPS % 2 == 0

_mesh = plsc.VectorSubcoreMesh(
    core_axis_name="c", subcore_axis_name="s", num_cores=NC, num_subcores=NS
)


@functools.partial(
    pl.kernel,
    out_type=jax.ShapeDtypeStruct((NUM_ROWS, DIM), jnp.float32),
    mesh=_mesh,
    compiler_params=pltpu.CompilerParams(use_tc_tiling_on_sc=False),
    scratch_types=[
        pltpu.VMEM((N_CHUNKS, CH), jnp.int32),
        pltpu.VMEM((GROUP, DIM), jnp.float32),
        pltpu.VMEM((GROUP, DIM), jnp.float32),
        pltpu.SemaphoreType.DMA,
        pltpu.SemaphoreType.DMA,
        pltpu.SemaphoreType.DMA,
        pltpu.SemaphoreType.DMA,
    ],
)
def _sc_gather(w_hbm, idx_hbm, out_hbm, idx_v, rows0, rows1, gs0, gs1, os0, os1):
    wid = lax.axis_index("s") * NC + lax.axis_index("c")
    base = wid * B_PER_W
    pltpu.sync_copy(idx_hbm.at[wid], idx_v)

    def fire(g, rows_v, sem):
        for j in range(K):
            pltpu.async_copy(
                w_hbm.at[idx_v.at[g * K + j]], rows_v.at[pl.ds(j * CH, CH)], sem
            )

    def wait_gathers(rows_v, sem):
        for j in range(K):
            pltpu.make_async_copy(
                w_hbm.at[idx_v.at[j]], rows_v.at[pl.ds(j * CH, CH)], sem
            ).wait()

    def start_out(g, rows_v, sem):
        pltpu.async_copy(rows_v, out_hbm.at[pl.ds(base + g * GROUP, GROUP)], sem)

    def wait_out(rows_v, sem):
        pltpu.make_async_copy(rows_v, out_hbm.at[pl.ds(base, GROUP)], sem).wait()

    fire(0, rows0, gs0)

    @pl.loop(0, N_GROUPS, step=2)
    def _pair(g):
        @pl.when(g > 0)
        def _():
            wait_out(rows1, os1)
        fire(g + 1, rows1, gs1)
        wait_gathers(rows0, gs0)
        start_out(g, rows0, os0)

        @pl.when(g + 2 < N_GROUPS)
        def _():
            wait_out(rows0, os0)
            fire(g + 2, rows0, gs0)
        wait_gathers(rows1, gs1)
        start_out(g + 1, rows1, os1)

    wait_out(rows0, os0)
    wait_out(rows1, os1)


def kernel(ids, weight):
    idx = ids.T.astype(jnp.int32).reshape(NW, N_CHUNKS, CH)
    out = _sc_gather(weight, idx)
    return out.reshape(NT, NB, DIM).transpose(1, 0, 2)
